# baseline (speedup 1.0000x reference)
import functools

import jax
import jax.numpy as jnp
from jax.experimental import pallas as pl


def _matmul_body(f_ref, w_ref, b_ref, o_ref):
    k = pl.program_id(0)
    L = w_ref.shape[0]
    D = w_ref.shape[2]
    for l in range(L):
        acc = jnp.dot(f_ref[...], w_ref[l], preferred_element_type=jnp.float32)
        col = pl.ds(l * D, D)

        @pl.when(k == 0)
        def _init():
            o_ref[:, col] = acc + b_ref[:, col]

        @pl.when(k != 0)
        def _accum():
            o_ref[:, col] += acc


@functools.partial(jax.jit, static_argnames=())
def kernel(f, weight, bias):
    B, F = f.shape
    L, _, D = weight.shape
    KBLK = 1024
    nk = F // KBLK

    out = pl.pallas_call(
        _matmul_body,
        grid=(nk,),
        in_specs=[
            pl.BlockSpec((B, KBLK), lambda k: (0, k)),
            pl.BlockSpec((L, KBLK, D), lambda k: (0, k, 0)),
            pl.BlockSpec((1, L * D), lambda k: (0, 0)),
        ],
        out_specs=pl.BlockSpec((B, L * D), lambda k: (0, 0)),
        out_shape=jax.ShapeDtypeStruct((B, L * D), jnp.float32),
    )(f, weight, bias.reshape(1, L * D))
    return out.reshape(B, L, D)


# speedup vs baseline: 1.0032x; 1.0032x over previous
import functools

import jax
import jax.numpy as jnp
from jax.experimental import pallas as pl
from jax.experimental.pallas import tpu as pltpu


def _matmul_body(f_ref, w_ref, b_ref, o_ref):
    k = pl.program_id(0)
    L = w_ref.shape[0]
    D = w_ref.shape[2]
    for l in range(L):
        acc = jnp.dot(f_ref[...], w_ref[l], preferred_element_type=jnp.float32)
        col = pl.ds(l * D, D)

        @pl.when(k == 0)
        def _init():
            o_ref[:, col] = acc + b_ref[:, col]

        @pl.when(k != 0)
        def _accum():
            o_ref[:, col] += acc


@functools.partial(jax.jit, static_argnames=())
def kernel(f, weight, bias):
    B, F = f.shape
    L, _, D = weight.shape
    KBLK = 4096
    nk = F // KBLK

    out = pl.pallas_call(
        _matmul_body,
        grid=(nk,),
        in_specs=[
            pl.BlockSpec((B, KBLK), lambda k: (0, k)),
            pl.BlockSpec((L, KBLK, D), lambda k: (0, k, 0)),
            pl.BlockSpec((1, L * D), lambda k: (0, 0)),
        ],
        out_specs=pl.BlockSpec((B, L * D), lambda k: (0, 0)),
        out_shape=jax.ShapeDtypeStruct((B, L * D), jnp.float32),
        compiler_params=pltpu.CompilerParams(
            vmem_limit_bytes=100 * 1024 * 1024),
    )(f, weight, bias.reshape(1, L * D))
    return out.reshape(B, L, D)


# speedup vs baseline: 1.0362x; 1.0329x over previous
import functools

import jax
import jax.numpy as jnp
from jax.experimental import pallas as pl


def _matmul_body(f_ref, w0_ref, w1_ref, b_ref, o_ref):
    k = pl.program_id(0)
    D = w0_ref.shape[2]
    for l, w_ref in enumerate((w0_ref, w1_ref)):
        acc = jnp.dot(f_ref[...], w_ref[0], preferred_element_type=jnp.float32)
        col = pl.ds(l * D, D)

        @pl.when(k == 0)
        def _init():
            o_ref[:, col] = acc + b_ref[:, col]

        @pl.when(k != 0)
        def _accum():
            o_ref[:, col] += acc


@functools.partial(jax.jit, static_argnames=())
def kernel(f, weight, bias):
    B, F = f.shape
    L, _, D = weight.shape
    KBLK = 2048
    nk = F // KBLK

    out = pl.pallas_call(
        _matmul_body,
        grid=(nk,),
        in_specs=[
            pl.BlockSpec((B, KBLK), lambda k: (0, k)),
            pl.BlockSpec((1, KBLK, D), lambda k: (0, k, 0)),
            pl.BlockSpec((1, KBLK, D), lambda k: (1, k, 0)),
            pl.BlockSpec((1, L * D), lambda k: (0, 0)),
        ],
        out_specs=pl.BlockSpec((B, L * D), lambda k: (0, 0)),
        out_shape=jax.ShapeDtypeStruct((B, L * D), jnp.float32),
    )(f, weight, weight, bias.reshape(1, L * D))
    return out.reshape(B, L, D)


# speedup vs baseline: 1.0489x; 1.0123x over previous
import functools

import jax
import jax.numpy as jnp
from jax.experimental import pallas as pl


def _matmul_body(f_ref, w_ref, b_ref, o_ref):
    k = pl.program_id(0)
    L = w_ref.shape[0]
    D = w_ref.shape[2]
    for l in range(L):
        acc = jnp.dot(f_ref[...], w_ref[l], preferred_element_type=jnp.float32)
        col = pl.ds(l * D, D)

        @pl.when(k == 0)
        def _init():
            o_ref[:, col] = acc + b_ref[:, col]

        @pl.when(k != 0)
        def _accum():
            o_ref[:, col] += acc


@functools.partial(jax.jit, static_argnames=())
def kernel(f, weight, bias):
    B, F = f.shape
    L, _, D = weight.shape
    KBLK = 2048
    nk = F // KBLK

    out = pl.pallas_call(
        _matmul_body,
        grid=(nk,),
        in_specs=[
            pl.BlockSpec((B, KBLK), lambda k: (0, k)),
            pl.BlockSpec((L, KBLK, D), lambda k: (0, k, 0)),
            pl.BlockSpec((1, L * D), lambda k: (0, 0)),
        ],
        out_specs=pl.BlockSpec((B, L * D), lambda k: (0, 0)),
        out_shape=jax.ShapeDtypeStruct((B, L * D), jnp.float32),
    )(f, weight, bias.reshape(1, L * D))
    return out.reshape(B, L, D)
